# Initial kernel scaffold; baseline (speedup 1.0000x reference)
import jax
import jax.numpy as jnp
from jax.experimental import pallas as pl


def kernel(x, memory, classes):
    raise NotImplementedError("write your pallas kernel here")


# speedup vs baseline: 144.9985x; 144.9985x over previous
import jax
import jax.numpy as jnp
from jax.experimental import pallas as pl

B = 1024
D = 128
C = 256
CAP = 64
TAUC = 1.0
CK = 32
NCHUNK = C // CK
W = CK * CAP

_f32 = jnp.float32


def _loss_kernel(x_ref, mem_ref, cls_ref, out_ref):
    x = x_ref[:, :]
    cls_col = cls_ref[:, :]

    nrm = jnp.sqrt(jnp.sum(x * x, axis=1, keepdims=True))
    feat = x / jnp.maximum(nrm, 1e-12)

    cls_iota = jax.lax.broadcasted_iota(jnp.int32, (B, C), 1)
    onehot = (cls_col == cls_iota).astype(_f32)
    counts_row = jnp.sum(onehot, axis=0, keepdims=True)

    ii = jax.lax.broadcasted_iota(jnp.int32, (B, B), 0)
    jj = jax.lax.broadcasted_iota(jnp.int32, (B, B), 1)
    lt = (jj < ii).astype(_f32)
    cex = jax.lax.dot_general(lt, onehot, (((1,), (0,)), ((), ())),
                              preferred_element_type=_f32)
    r_col = jnp.sum(cex * onehot, axis=1, keepdims=True)
    incl_col = (r_col < float(CAP)).astype(_f32)

    ident = (jj == ii).astype(_f32)
    incl_row = jax.lax.dot_general(incl_col, ident, (((0,), (0,)), ((), ())),
                                   preferred_element_type=_f32)

    same = jax.lax.dot_general(onehot, onehot, (((1,), (1,)), ((), ())),
                               preferred_element_type=_f32)

    h_col = (cls_col // 64).astype(_f32)
    p_col = (cls_col - (cls_col // 64) * 64).astype(_f32)
    cm4_col = (cls_col - (cls_col // 4) * 4).astype(_f32)
    cm4_row = jax.lax.dot_general(cm4_col, ident, (((0,), (0,)), ((), ())),
                                  preferred_element_type=_f32)
    r_row = jax.lax.dot_general(r_col, ident, (((0,), (0,)), ((), ())),
                                preferred_element_type=_f32)
    ex_new = ((cm4_row == h_col) & (r_row == p_col)).astype(_f32)

    G = jax.lax.dot_general(feat, x, (((1,), (1,)), ((), ())),
                            preferred_element_type=_f32) * (1.0 / TAUC)

    m = jnp.max(G, axis=1, keepdims=True)
    T = jnp.zeros((B, 1), _f32)
    posM = jnp.zeros((B, 1), _f32)

    for k in range(NCHUNK):
        mb = mem_ref[pl.ds(k * W, W), :]
        Mc = jax.lax.dot_general(feat, mb, (((1,), (1,)), ((), ())),
                                 preferred_element_type=_f32) * (1.0 / TAUC)

        col = jax.lax.broadcasted_iota(jnp.int32, (1, W), 1)
        lcls = col // CAP
        t_id = col - lcls * CAP
        gcls = lcls + (k * CK)
        excl = (gcls == cls_col).astype(_f32)

        countsc = counts_row[:, k * CK:(k + 1) * CK]
        oc = (jax.lax.broadcasted_iota(jnp.int32, (W, CK), 0) // CAP ==
              jax.lax.broadcasted_iota(jnp.int32, (W, CK), 1)).astype(_f32)
        ccol = jax.lax.dot_general(countsc, oc, (((1,), (1,)), ((), ())),
                                   preferred_element_type=_f32)
        keep = (t_id.astype(_f32) + ccol < float(CAP)).astype(_f32)

        gcls_m4 = (gcls - (gcls // 4) * 4).astype(_f32)
        tpc = t_id.astype(_f32) + ccol
        ex_old = ((gcls_m4 == h_col) & (tpc == p_col)).astype(_f32)

        me = jnp.maximum(m, jnp.max(Mc, axis=1, keepdims=True))
        scale = jnp.exp(m - me)
        eMk = jnp.exp(Mc - me) * keep
        T = T * scale + jnp.sum(eMk * (1.0 - ex_old), axis=1, keepdims=True)
        posM = posM + jnp.sum(Mc * keep * excl, axis=1, keepdims=True)
        m = me

    eG = jnp.exp(G - m) * incl_row
    T = T + jnp.sum(eG * (1.0 - ex_new), axis=1, keepdims=True)
    posG = jnp.sum(G * incl_row * same, axis=1, keepdims=True)

    pos = (posM + posG) * (1.0 / CAP)
    denom = jnp.exp(pos - m) + T
    lossv = (m - pos) + jnp.log(denom)
    out_ref[:, :] = jnp.reshape(jnp.sum(lossv) * (1.0 / B), (1, 1))


def kernel(x, memory, classes):
    mem_flat = memory.reshape(C * CAP, D)
    cls2d = classes.reshape(B, 1)
    out = pl.pallas_call(
        _loss_kernel,
        out_shape=jax.ShapeDtypeStruct((1, 1), jnp.float32),
    )(x, mem_flat, cls2d)
    return out[0, 0]
